# baseline (speedup 1.0000x reference)
import jax
import jax.numpy as jnp
from jax.experimental import pallas as pl
from jax.experimental.pallas import tpu as pltpu

_H, _W = 50, 38
_Q = _H * _W
_QP = 1920
_C = 512
_NL, _NS = 36, 18
_NH = _NL + _NS
_MARG = 64
_SFH = 2048


def _body(x_ref, w_ref, lw_ref, sw_ref, b1_ref, lb_ref, sb_ref,
          locT_ref, scoT_ref, sf_ref, wbuf_ref, sem):
    pltpu.make_async_copy(w_ref.at[0], wbuf_ref.at[0], sem.at[0]).start()

    sf_ref[:_MARG, :] = jnp.zeros((_MARG, _C), jnp.bfloat16)
    sf_ref[_MARG + _Q:, :] = jnp.zeros((_SFH - _MARG - _Q, _C), jnp.bfloat16)
    for g in range(4):
        colv = x_ref[pl.ds(g, _Q, 4), :]
        sf_ref[_MARG:_MARG + _Q, 128 * g:128 * (g + 1)] = (
            colv.astype(jnp.bfloat16))

    h = jnp.concatenate([lw_ref[:], sw_ref[:]], axis=0)
    hb = jnp.concatenate([lb_ref[:], sb_ref[:]], axis=1)
    bias = jnp.sum(h.astype(jnp.float32) * b1_ref[:],
                   axis=1, keepdims=True).T + hb

    q = jax.lax.broadcasted_iota(jnp.int32, (_QP, 1), 0)
    wcol = q - (q // _W) * _W
    mask_l = (wcol > 0).astype(jnp.float32)
    mask_r = (wcol < _W - 1).astype(jnp.float32)

    sfb = sf_ref[:, :]
    acc = jnp.zeros((_QP, _NH), jnp.float32)
    for ky in range(3):
        for kx in range(3):
            t = ky * 3 + kx
            delta = (ky - 1) * _W + (kx - 1)
            if t + 1 < 9:
                pltpu.make_async_copy(w_ref.at[t + 1],
                                      wbuf_ref.at[(t + 1) % 2],
                                      sem.at[(t + 1) % 2]).start()
            pltpu.make_async_copy(w_ref.at[t], wbuf_ref.at[t % 2],
                                  sem.at[t % 2]).wait()
            cwwT = jnp.dot(h, wbuf_ref[t % 2],
                           preferred_element_type=jnp.float32).T
            p = jnp.dot(sfb, cwwT.astype(jnp.bfloat16),
                        preferred_element_type=jnp.float32)
            contr = p[_MARG + delta:_MARG + delta + _QP, :]
            if kx == 0:
                contr = contr * mask_l
            elif kx == 2:
                contr = contr * mask_r
            acc = acc + contr
    acc = acc + bias
    locT_ref[:] = acc[:_Q, :_NL]
    scoT_ref[:] = acc[:_Q, _NL:]


def kernel(out_map, conv1_w, conv1_b, loc_w, loc_b, score_w, score_b):
    xT = out_map.transpose(2, 3, 0, 1).reshape(_Q * 4, _C // 4)
    w9 = conv1_w.transpose(2, 3, 0, 1).reshape(9, _C, _C)
    lw = loc_w.transpose(0, 2, 3, 1).reshape(_NL, _C).astype(jnp.bfloat16)
    sw = score_w.transpose(0, 2, 3, 1).reshape(_NS, _C).astype(jnp.bfloat16)
    b1 = conv1_b.reshape(1, _C)
    lb = loc_b.reshape(1, _NL)
    sb = score_b.reshape(1, _NS)

    locT, scoT = pl.pallas_call(
        _body,
        in_specs=[
            pl.BlockSpec(memory_space=pltpu.VMEM),
            pl.BlockSpec(memory_space=pl.ANY),
            pl.BlockSpec(memory_space=pltpu.VMEM),
            pl.BlockSpec(memory_space=pltpu.VMEM),
            pl.BlockSpec(memory_space=pltpu.VMEM),
            pl.BlockSpec(memory_space=pltpu.VMEM),
            pl.BlockSpec(memory_space=pltpu.VMEM),
        ],
        out_shape=(jax.ShapeDtypeStruct((_Q, _NL), jnp.float32),
                   jax.ShapeDtypeStruct((_Q, _NS), jnp.float32)),
        scratch_shapes=[
            pltpu.VMEM((_SFH, _C), jnp.bfloat16),
            pltpu.VMEM((2, _C, _C), jnp.float32),
            pltpu.SemaphoreType.DMA((2,)),
        ],
    )(xT, w9, lw, sw, b1, lb, sb)

    loc = locT.reshape(_H, _W, _NL).transpose(2, 0, 1)[None]
    score = scoT.reshape(_H, _W, _NS).transpose(2, 0, 1)[None]
    return (loc, score)


# speedup vs baseline: 1.0050x; 1.0050x over previous
import jax
import jax.numpy as jnp
from jax.experimental import pallas as pl
from jax.experimental.pallas import tpu as pltpu

_H, _W = 50, 38
_Q = _H * _W
_QP = 1920
_C = 512
_NL, _NS = 36, 18
_NH = _NL + _NS
_MARG = 64
_SFH = 2048


def _body(x_ref, w_ref, lw_ref, sw_ref, b1_ref, lb_ref, sb_ref,
          locT_ref, scoT_ref, sf_ref):
    sf_ref[:_MARG, :] = jnp.zeros((_MARG, _C), jnp.bfloat16)
    sf_ref[_MARG + _Q:, :] = jnp.zeros((_SFH - _MARG - _Q, _C), jnp.bfloat16)
    for g in range(4):
        colv = x_ref[pl.ds(g, _Q, 4), :]
        sf_ref[_MARG:_MARG + _Q, 128 * g:128 * (g + 1)] = (
            colv.astype(jnp.bfloat16))

    h = jnp.concatenate([lw_ref[:], sw_ref[:]], axis=0)
    hb = jnp.concatenate([lb_ref[:], sb_ref[:]], axis=1)
    bias = jnp.sum(h.astype(jnp.float32) * b1_ref[:],
                   axis=1, keepdims=True).T + hb

    q = jax.lax.broadcasted_iota(jnp.int32, (_QP, 1), 0)
    wcol = q - (q // _W) * _W
    mask_l = (wcol > 0).astype(jnp.float32)
    mask_r = (wcol < _W - 1).astype(jnp.float32)

    sfb = sf_ref[:, :]
    acc = jnp.zeros((_QP, _NH), jnp.float32)
    for ky in range(3):
        for kx in range(3):
            t = ky * 3 + kx
            delta = (ky - 1) * _W + (kx - 1)
            cwwT = jnp.dot(h, w_ref[t],
                           preferred_element_type=jnp.float32).T
            p = jnp.dot(sfb, cwwT.astype(jnp.bfloat16),
                        preferred_element_type=jnp.float32)
            contr = p[_MARG + delta:_MARG + delta + _QP, :]
            if kx == 0:
                contr = contr * mask_l
            elif kx == 2:
                contr = contr * mask_r
            acc = acc + contr
    acc = acc + bias
    locT_ref[:] = acc[:_Q, :_NL]
    scoT_ref[:] = acc[:_Q, _NL:]


def kernel(out_map, conv1_w, conv1_b, loc_w, loc_b, score_w, score_b):
    xT = out_map.transpose(2, 3, 0, 1).reshape(_Q * 4, _C // 4)
    w9 = conv1_w.transpose(2, 3, 0, 1).reshape(9, _C, _C).astype(jnp.bfloat16)
    lw = loc_w.transpose(0, 2, 3, 1).reshape(_NL, _C).astype(jnp.bfloat16)
    sw = score_w.transpose(0, 2, 3, 1).reshape(_NS, _C).astype(jnp.bfloat16)
    b1 = conv1_b.reshape(1, _C)
    lb = loc_b.reshape(1, _NL)
    sb = score_b.reshape(1, _NS)

    locT, scoT = pl.pallas_call(
        _body,
        out_shape=(jax.ShapeDtypeStruct((_Q, _NL), jnp.float32),
                   jax.ShapeDtypeStruct((_Q, _NS), jnp.float32)),
        scratch_shapes=[
            pltpu.VMEM((_SFH, _C), jnp.bfloat16),
        ],
    )(xT, w9, lw, sw, b1, lb, sb)

    loc = locT.reshape(_H, _W, _NL).transpose(2, 0, 1)[None]
    score = scoT.reshape(_H, _W, _NS).transpose(2, 0, 1)[None]
    return (loc, score)


# speedup vs baseline: 1.2166x; 1.2106x over previous
import jax
import jax.numpy as jnp
from jax.experimental import pallas as pl
from jax.experimental.pallas import tpu as pltpu

_H, _W = 50, 38
_Q = _H * _W
_QP = 1920
_C = 512
_NL, _NS = 36, 18
_NH = _NL + _NS
_MARG = 64
_SFH = 2048


def _body(x_ref, w_ref, lw_ref, sw_ref, b1_ref, lb_ref, sb_ref,
          locT_ref, scoT_ref, sf_ref):
    sf_ref[:_MARG, :] = jnp.zeros((_MARG, _C), jnp.bfloat16)
    sf_ref[_MARG + _Q:, :] = jnp.zeros((_SFH - _MARG - _Q, _C), jnp.bfloat16)
    for g in range(4):
        colv = x_ref[pl.ds(g, _Q, 4), :]
        sf_ref[_MARG:_MARG + _Q, 128 * g:128 * (g + 1)] = (
            colv.astype(jnp.bfloat16))

    h = jnp.concatenate([lw_ref[:], sw_ref[:]], axis=0)
    hb = jnp.concatenate([lb_ref[:], sb_ref[:]], axis=1)
    bias = jnp.sum(h.astype(jnp.float32) * b1_ref[:],
                   axis=1, keepdims=True).T + hb

    q = jax.lax.broadcasted_iota(jnp.int32, (_QP, 1), 0)
    wcol = q - (q // _W) * _W
    mask_l = (wcol > 0).astype(jnp.float32)
    mask_r = (wcol < _W - 1).astype(jnp.float32)

    sfb = sf_ref[:, :]
    acc = jnp.zeros((_QP, _NH), jnp.float32)
    for ky in range(3):
        for kx in range(3):
            t = ky * 3 + kx
            delta = (ky - 1) * _W + (kx - 1)
            cwwT = jnp.dot(h, w_ref[t],
                           preferred_element_type=jnp.float32).T
            p = jnp.dot(sfb, cwwT.astype(jnp.bfloat16),
                        preferred_element_type=jnp.float32)
            contr = p[_MARG + delta:_MARG + delta + _QP, :]
            if kx == 0:
                contr = contr * mask_l
            elif kx == 2:
                contr = contr * mask_r
            acc = acc + contr
    acc = acc + bias
    locT_ref[:] = acc[:_Q, :_NL]
    scoT_ref[:] = acc[:_Q, _NL:]


def kernel(out_map, conv1_w, conv1_b, loc_w, loc_b, score_w, score_b):
    xT = out_map.transpose(2, 3, 0, 1).reshape(_Q * 4, _C // 4)
    w9 = conv1_w.transpose(2, 3, 0, 1).reshape(9, _C, _C)
    lw = loc_w.transpose(0, 2, 3, 1).reshape(_NL, _C).astype(jnp.bfloat16)
    sw = score_w.transpose(0, 2, 3, 1).reshape(_NS, _C).astype(jnp.bfloat16)
    b1 = conv1_b.reshape(1, _C)
    lb = loc_b.reshape(1, _NL)
    sb = score_b.reshape(1, _NS)

    locT, scoT = pl.pallas_call(
        _body,
        out_shape=(jax.ShapeDtypeStruct((_Q, _NL), jnp.float32),
                   jax.ShapeDtypeStruct((_Q, _NS), jnp.float32)),
        scratch_shapes=[
            pltpu.VMEM((_SFH, _C), jnp.bfloat16),
        ],
    )(xT, w9, lw, sw, b1, lb, sb)

    loc = locT.reshape(_H, _W, _NL).transpose(2, 0, 1)[None]
    score = scoT.reshape(_H, _W, _NS).transpose(2, 0, 1)[None]
    return (loc, score)
